# baseline (speedup 1.0000x reference)
import functools

import numpy as np
import jax
import jax.numpy as jnp
from jax import lax
from jax.experimental import pallas as pl
from jax.experimental.pallas import tpu as pltpu, tpu_sc as plsc

D_MODEL = 64
SEQ_LEN = 8192
LANES = 16


def _positional_encoding_np(seq_len: int, d_model: int) -> np.ndarray:
    pos = np.arange(seq_len, dtype=np.float32)[:, None]
    div_term = np.exp(
        np.arange(0, d_model, 2, dtype=np.float32) * (-np.log(10000.0) / d_model)
    )
    pe = np.zeros((seq_len, d_model), dtype=np.float32)
    pe[:, 0::2] = np.sin(pos * div_term)
    pe[:, 1::2] = np.cos(pos * div_term)
    return pe


_PE = _positional_encoding_np(SEQ_LEN, D_MODEL)


@functools.cache
def _build_sc_kernel(vocab: int, seq_len: int, d_model: int):
    mesh = plsc.VectorSubcoreMesh(core_axis_name="c", subcore_axis_name="s")
    nw = mesh.num_cores * mesh.num_subcores
    b_per_w = seq_len // nw
    n_idx_rows = b_per_w // 128

    @functools.partial(
        pl.kernel,
        mesh=mesh,
        out_type=jax.ShapeDtypeStruct((seq_len, d_model), jnp.float32),
        scratch_types=[
            pltpu.VMEM((n_idx_rows, 128), jnp.int32),
            pltpu.VMEM((b_per_w, d_model), jnp.float32),
            pltpu.VMEM((b_per_w, d_model), jnp.float32),
            pltpu.SemaphoreType.DMA,
        ],
        compiler_params=pltpu.CompilerParams(use_tc_tiling_on_sc=False),
    )
    def sc_embed(emb_hbm, idx_hbm, pe_hbm, out_hbm, idx_v, rows_v, pe_v, sem):
        wid = lax.axis_index("s") * mesh.num_cores + lax.axis_index("c")
        base = wid * b_per_w
        pltpu.sync_copy(idx_hbm.at[pl.ds(wid * n_idx_rows, n_idx_rows)], idx_v)
        copies = [
            pltpu.async_copy(
                emb_hbm.at[idx_v.at[j]],
                rows_v.at[pl.ds(j * 128, 128)],
                sem,
            )
            for j in range(n_idx_rows)
        ]
        pltpu.sync_copy(pe_hbm.at[pl.ds(base, b_per_w)], pe_v)
        for c in copies:
            c.wait()

        chunks = d_model // LANES

        def add_row(r, _):
            for c in range(chunks):
                plsc.addupdate(
                    rows_v.at[r, pl.ds(c * LANES, LANES)],
                    pe_v[r, pl.ds(c * LANES, LANES)],
                )
            return _

        lax.fori_loop(0, b_per_w, add_row, None, unroll=4)

        pltpu.sync_copy(rows_v, out_hbm.at[pl.ds(base, b_per_w)])

    return sc_embed


def kernel(token_ids, emb):
    vocab, d_model = emb.shape
    seq_len = token_ids.shape[0]
    ids = token_ids.astype(jnp.int32).reshape(seq_len // 128, 128)
    pe = jnp.asarray(_PE)
    out = _build_sc_kernel(vocab, seq_len, d_model)(emb, ids, pe)
    return out[None, :, :]


# speedup vs baseline: 1.9175x; 1.9175x over previous
import functools

import numpy as np
import jax
import jax.numpy as jnp
from jax import lax
from jax.experimental import pallas as pl
from jax.experimental.pallas import tpu as pltpu, tpu_sc as plsc

D_MODEL = 64
SEQ_LEN = 8192
LANES = 16


def _positional_encoding_np(seq_len: int, d_model: int) -> np.ndarray:
    pos = np.arange(seq_len, dtype=np.float32)[:, None]
    div_term = np.exp(
        np.arange(0, d_model, 2, dtype=np.float32) * (-np.log(10000.0) / d_model)
    )
    pe = np.zeros((seq_len, d_model), dtype=np.float32)
    pe[:, 0::2] = np.sin(pos * div_term)
    pe[:, 1::2] = np.cos(pos * div_term)
    return pe


_PE_T = np.ascontiguousarray(_positional_encoding_np(SEQ_LEN, D_MODEL).T).reshape(-1)


@functools.cache
def _build_sc_kernel(vocab: int, seq_len: int, d_model: int):
    mesh = plsc.VectorSubcoreMesh(core_axis_name="c", subcore_axis_name="s")
    nw = mesh.num_cores * mesh.num_subcores
    rows_per_w = d_model // nw
    bulk = (vocab // 128) * 128
    tail = vocab - bulk

    @functools.partial(
        pl.kernel,
        mesh=mesh,
        out_type=jax.ShapeDtypeStruct((d_model, seq_len), jnp.float32),
        scratch_types=[
            pltpu.VMEM((seq_len,), jnp.int32),
            pltpu.VMEM((vocab,), jnp.float32),
            pltpu.VMEM((seq_len,), jnp.float32),
            pltpu.VMEM((seq_len,), jnp.float32),
        ],
        compiler_params=pltpu.CompilerParams(
            use_tc_tiling_on_sc=True, needs_layout_passes=False
        ),
    )
    def sc_embed(
        emb_t_hbm, ids_hbm, pe_hbm, tail_hbm, out_hbm, idx_v, row_v, pe_v, out_v
    ):
        wid = lax.axis_index("s") * mesh.num_cores + lax.axis_index("c")
        pltpu.sync_copy(ids_hbm, idx_v)

        for rr in range(rows_per_w):
            d = wid * rows_per_w + rr
            pltpu.sync_copy(
                emb_t_hbm.at[d, pl.ds(0, bulk)], row_v.at[pl.ds(0, bulk)]
            )
            if tail:
                pltpu.sync_copy(
                    tail_hbm.at[pl.ds(d * tail, tail)], row_v.at[pl.ds(bulk, tail)]
                )
            pltpu.sync_copy(pe_hbm.at[pl.ds(d * seq_len, seq_len)], pe_v)

            def gather_chunk(i, _):
                sl = pl.ds(i * LANES, LANES)
                ids16 = idx_v[sl]
                vals = plsc.load_gather(row_v, [ids16])
                out_v[sl] = vals + pe_v[sl]
                return _

            lax.fori_loop(0, seq_len // LANES, gather_chunk, None, unroll=8)

            pltpu.sync_copy(out_v, out_hbm.at[d, pl.ds(0, seq_len)])

    return sc_embed


def kernel(token_ids, emb):
    vocab, d_model = emb.shape
    seq_len = token_ids.shape[0]
    ids = token_ids.astype(jnp.int32)
    pe_t = jnp.asarray(_PE_T)
    bulk = (vocab // 128) * 128
    emb_tail = emb.T[:, bulk:].reshape(-1)
    out_t = _build_sc_kernel(vocab, seq_len, d_model)(emb.T, ids, pe_t, emb_tail)
    return out_t.T[None, :, :]
